# baseline (speedup 1.0000x reference)
import functools

import jax
import jax.numpy as jnp
from jax import lax
from jax.experimental import pallas as pl
from jax.experimental.pallas import tpu as pltpu

H = 128


def _mlp_body(x_ref, w1_ref, b1_ref, w2_ref, b2_ref, w3_ref, b3_ref, o_ref):
    x = x_ref[...]
    h = jnp.maximum(jnp.dot(x, w1_ref[...], preferred_element_type=jnp.float32) + b1_ref[...], 0.0)
    h = jnp.maximum(jnp.dot(h, w2_ref[...], preferred_element_type=jnp.float32) + b2_ref[...], 0.0)
    o_ref[...] = jnp.dot(h, w3_ref[...], preferred_element_type=jnp.float32) + b3_ref[...]


def _mlp(x, p, name, blk=1000):
    n = x.shape[0]
    d = x.shape[1]
    w1 = p["mlp_%s_W1" % name]; b1 = p["mlp_%s_b1" % name].reshape(1, H)
    w2 = p["mlp_%s_W2" % name]; b2 = p["mlp_%s_b2" % name].reshape(1, H)
    w3 = p["mlp_%s_W3" % name]; b3 = p["mlp_%s_b3" % name].reshape(1, H)
    full = lambda r, c: pl.BlockSpec((r, c), lambda i: (0, 0))
    return pl.pallas_call(
        _mlp_body,
        grid=(n // blk,),
        in_specs=[
            pl.BlockSpec((blk, d), lambda i: (i, 0)),
            full(d, H), full(1, H), full(H, H), full(1, H), full(H, H), full(1, H),
        ],
        out_specs=pl.BlockSpec((blk, H), lambda i: (i, 0)),
        out_shape=jax.ShapeDtypeStruct((n, H), jnp.float32),
    )(x, w1, b1, w2, b2, w3, b3)


def _combine2_body(s1_ref, c1_ref, s2_ref, c2_ref, x_ref, wl1_ref, wl2_ref,
                   wr1_ref, wr2_ref, b_ref, o_ref):
    inv1 = 1.0 / jnp.maximum(c1_ref[:, 0:1], 1.0)
    inv2 = 1.0 / jnp.maximum(c2_ref[:, 0:1], 1.0)
    m1 = s1_ref[...] * inv1
    m2 = s2_ref[...] * inv2
    acc = jnp.dot(m1, wl1_ref[...], preferred_element_type=jnp.float32)
    acc += jnp.dot(m2, wl2_ref[...], preferred_element_type=jnp.float32)
    acc += jnp.dot(x_ref[...], wr1_ref[...] + wr2_ref[...], preferred_element_type=jnp.float32)
    o_ref[...] = jnp.maximum(acc + b_ref[...], 0.0)


def _combine2(s1, c1, s2, c2, x, wl1, wl2, wr1, wr2, b, blk=1000):
    n = x.shape[0]
    full = lambda r, c: pl.BlockSpec((r, c), lambda i: (0, 0))
    row = lambda c: pl.BlockSpec((blk, c), lambda i: (i, 0))
    return pl.pallas_call(
        _combine2_body,
        grid=(n // blk,),
        in_specs=[row(H), row(16), row(H), row(16), row(H),
                  full(H, H), full(H, H), full(H, H), full(H, H), full(1, H)],
        out_specs=row(H),
        out_shape=jax.ShapeDtypeStruct((n, H), jnp.float32),
    )(s1, c1, s2, c2, x, wl1, wl2, wr1, wr2, b)


def _combine1_body(s_ref, c_ref, x_ref, wl_ref, wr_ref, b_ref, o_ref):
    inv = 1.0 / jnp.maximum(c_ref[:, 0:1], 1.0)
    acc = jnp.dot(s_ref[...] * inv, wl_ref[...], preferred_element_type=jnp.float32)
    acc += jnp.dot(x_ref[...], wr_ref[...], preferred_element_type=jnp.float32)
    o_ref[...] = jnp.maximum(acc + b_ref[...], 0.0)


def _combine1(s, c, x, wl, wr, b, blk=1000):
    n = x.shape[0]
    full = lambda r, cc: pl.BlockSpec((r, cc), lambda i: (0, 0))
    row = lambda cc: pl.BlockSpec((blk, cc), lambda i: (i, 0))
    return pl.pallas_call(
        _combine1_body,
        grid=(n // blk,),
        in_specs=[row(H), row(16), row(H), full(H, H), full(H, H), full(1, H)],
        out_specs=row(H),
        out_shape=jax.ShapeDtypeStruct((n, H), jnp.float32),
    )(s, c, x, wl, wr, b)


def _final_body(s_ref, c_ref, x_ref, wl_ref, wr_ref, b_ref, lw_ref, lb_ref,
                bs_ref, o_ref, *, blk):
    inv = 1.0 / jnp.maximum(c_ref[:, 0:1], 1.0)
    acc = jnp.dot(s_ref[...] * inv, wl_ref[...], preferred_element_type=jnp.float32)
    acc += jnp.dot(x_ref[...], wr_ref[...], preferred_element_type=jnp.float32)
    h = jnp.maximum(acc + b_ref[...], 0.0)
    o = jnp.maximum(jnp.dot(h, lw_ref[...], preferred_element_type=jnp.float32) + lb_ref[...], 0.0)
    rows = pl.program_id(0) * blk + lax.broadcasted_iota(jnp.int32, (blk, 1), 0)
    o_ref[...] = jnp.where(rows < bs_ref[0, 0], o, 0.0)


def _final(s, c, x, wl, wr, b, lw, lb, bs, blk=1000):
    n = x.shape[0]
    full = lambda r, cc: pl.BlockSpec((r, cc), lambda i: (0, 0))
    row = lambda cc: pl.BlockSpec((blk, cc), lambda i: (i, 0))
    return pl.pallas_call(
        functools.partial(_final_body, blk=blk),
        grid=(n // blk,),
        in_specs=[row(H), row(16), row(H), full(H, H), full(H, H), full(1, H),
                  full(H, 1), full(1, 1),
                  pl.BlockSpec(memory_space=pltpu.SMEM)],
        out_specs=row(1),
        out_shape=jax.ShapeDtypeStruct((n, 1), jnp.float32),
    )(s, c, x, wl, wr, b, lw, lb, bs)


def _segsum(x_src, ei, n_dst):
    msg = jnp.take(x_src, ei[0], axis=0)
    s = jax.ops.segment_sum(msg, ei[1], num_segments=n_dst)
    c = jax.ops.segment_sum(jnp.ones((ei.shape[1],), jnp.float32), ei[1], num_segments=n_dst)
    c16 = jnp.broadcast_to(c[:, None], (n_dst, 16))
    return s, c16


def kernel(x_SNP, x_Gene, x_CC, x_BP, x_MF, ei_snp_gene, ei_gene_snp,
           ei_gene_gene, ei_gene_cc, ei_gene_bp, ei_gene_mf, params, batch_size):
    p = params
    n_snp = x_SNP.shape[0]
    n_gene = x_Gene.shape[0]

    h_snp = _mlp(x_SNP, p, "snp")
    h_gene = _mlp(x_Gene, p, "gene")

    s_sg, c_sg = _segsum(h_snp, ei_snp_gene, n_gene)
    s_gg, c_gg = _segsum(h_gene, ei_gene_gene, n_gene)
    s_gs, c_gs = _segsum(h_gene, ei_gene_snp, n_snp)

    gene1 = _combine2(
        s_sg, c_sg, s_gg, c_gg, h_gene,
        p["conv0_snp_gene_Wl"], p["conv0_gene_gene_Wl"],
        p["conv0_snp_gene_Wr"], p["conv0_gene_gene_Wr"],
        (p["conv0_snp_gene_bl"] + p["conv0_gene_gene_bl"]).reshape(1, H))
    snp1 = _combine1(
        s_gs, c_gs, h_snp,
        p["conv0_gene_snp_Wl"], p["conv0_gene_snp_Wr"],
        p["conv0_gene_snp_bl"].reshape(1, H))

    s2_gs, _ = _segsum(gene1, ei_gene_snp, n_snp)

    bs = jnp.asarray(batch_size, jnp.int32).reshape(1, 1)
    out = _final(
        s2_gs, c_gs, snp1,
        p["conv1_gene_snp_Wl"], p["conv1_gene_snp_Wr"],
        p["conv1_gene_snp_bl"].reshape(1, H),
        p["lin_W"], p["lin_b"].reshape(1, 1), bs)
    return out
